# baseline (speedup 1.0000x reference)
import functools

import jax
import jax.numpy as jnp
from jax import lax
from jax.experimental import pallas as pl
from jax.experimental.pallas import tpu as pltpu
from jax.experimental.pallas import tpu_sc as plsc

_E = 8
_T = 2048
_K = 2
_TK = _T * _K
_D = 1024
_F = 2048
_SEG = _TK // _E
_BF = 1024
_NF = _F // _BF

_NC = 2
_NS = 16
_DH = _D // _NC
_RPT = _TK // _NS
_RB = 64
_NB = _RPT // _RB
_WPT = _T // _NS
_GPW = _TK // (_NC * _NS)

_MESH = plsc.VectorSubcoreMesh(core_axis_name="c", subcore_axis_name="s")


@functools.partial(
    pl.kernel,
    out_type=jax.ShapeDtypeStruct((_TK,), jnp.float32),
    mesh=_MESH,
    scratch_types=[
        pltpu.VMEM((_TK,), jnp.float32),
        pltpu.VMEM((_GPW,), jnp.int32),
        pltpu.VMEM((_GPW,), jnp.float32),
    ],
    compiler_params=pltpu.CompilerParams(needs_layout_passes=False),
)
def _gather_gates(gates_hbm, gidx_hbm, g_hbm, gates_v, gidx_v, gout_v):
    wid = lax.axis_index("s") * _NC + lax.axis_index("c")
    base = wid * _GPW
    pltpu.sync_copy(gates_hbm, gates_v)
    pltpu.sync_copy(gidx_hbm.at[pl.ds(base, _GPW)], gidx_v)
    for i in range(_GPW // 16):
        idx16 = gidx_v[pl.ds(i * 16, 16)]
        gout_v[pl.ds(i * 16, 16)] = plsc.load_gather(gates_v, [idx16])
    pltpu.sync_copy(gout_v, g_hbm.at[pl.ds(base, _GPW)])


def _mlp_body(offs_ref, x_ref, w_ref, b_ref, wp_ref, bp_ref, wo_ref, bo_ref,
              g_ref, y_ref):
    f = pl.program_id(1)
    x = x_ref[...].astype(jnp.bfloat16)
    xw = jnp.dot(x, w_ref[0].astype(jnp.bfloat16),
                 preferred_element_type=jnp.float32) + b_ref[0, 0]
    xwp = jnp.dot(x, wp_ref[0].astype(jnp.bfloat16),
                  preferred_element_type=jnp.float32) + bp_ref[0, 0]
    h = ((xw * jax.nn.sigmoid(xw)) * xwp).astype(jnp.bfloat16)
    part = jnp.dot(h, wo_ref[0].astype(jnp.bfloat16),
                   preferred_element_type=jnp.float32)

    @pl.when(f == 0)
    def _():
        y_ref[...] = part

    @pl.when(f > 0)
    def _():
        y_ref[...] = y_ref[...] + part

    @pl.when(f == _NF - 1)
    def _():
        y_ref[...] = (y_ref[...] + bo_ref[0, 0]) * g_ref[...]


def _mlp(offsets, jagged, weight, bias, weight_p, bias_p, weight_out,
         bias_out, g2d):
    grid_spec = pltpu.PrefetchScalarGridSpec(
        num_scalar_prefetch=1,
        grid=(_E, _NF),
        in_specs=[
            pl.BlockSpec((_SEG, _D), lambda e, f, offs: (offs[e] // _SEG, 0)),
            pl.BlockSpec((1, _D, _BF), lambda e, f, offs: (e, 0, f)),
            pl.BlockSpec((1, 1, _BF), lambda e, f, offs: (e, 0, f)),
            pl.BlockSpec((1, _D, _BF), lambda e, f, offs: (e, 0, f)),
            pl.BlockSpec((1, 1, _BF), lambda e, f, offs: (e, 0, f)),
            pl.BlockSpec((1, _BF, _D), lambda e, f, offs: (e, f, 0)),
            pl.BlockSpec((1, 1, _D), lambda e, f, offs: (e, 0, 0)),
            pl.BlockSpec((_SEG, 1), lambda e, f, offs: (e, 0)),
        ],
        out_specs=pl.BlockSpec((_SEG, _D), lambda e, f, offs: (e, 0)),
    )
    return pl.pallas_call(
        _mlp_body,
        grid_spec=grid_spec,
        out_shape=jax.ShapeDtypeStruct((_TK, _D), jnp.float32),
        compiler_params=pltpu.CompilerParams(
            dimension_semantics=("arbitrary", "arbitrary"),
            vmem_limit_bytes=100 * 1024 * 1024),
    )(offsets, jagged, weight, bias.reshape(_E, 1, _F), weight_p,
      bias_p.reshape(_E, 1, _F), weight_out, bias_out.reshape(_E, 1, _D), g2d)


_NW = _NC * _NS
_TW = _T // _NW
_CHK = 16
_LCAP = _TK + 4 * _CHK


@functools.partial(
    pl.kernel,
    out_type=jax.ShapeDtypeStruct((_T, _D), jnp.float32),
    mesh=_MESH,
    scratch_types=[
        pltpu.VMEM((_TW + 1, _D), jnp.float32),
        pltpu.VMEM((_TK,), jnp.int32),
        pltpu.VMEM((_LCAP,), jnp.int32),
        pltpu.VMEM((_LCAP,), jnp.int32),
        pltpu.VMEM((2, _CHK, _D), jnp.float32),
        pltpu.SemaphoreType.DMA,
        pltpu.SemaphoreType.DMA,
    ],
    compiler_params=pltpu.CompilerParams(
        needs_layout_passes=False, use_tc_tiling_on_sc=True),
)
def _scatter_combine(y_hbm, idx_hbm, out_hbm, acc, idx_v, rowlist, jlist, yb,
                     semA, semB):
    w = lax.axis_index("s") * _NC + lax.axis_index("c")
    lanes = jnp.arange(16, dtype=jnp.int32)
    sems = (semA, semB)

    def _zrow(r, carry):
        for u in range(_D // 16):
            acc[r, pl.ds(u * 16, 16)] = jnp.zeros((16,), jnp.float32)
        return carry

    lax.fori_loop(0, _TW + 1, _zrow, 0)

    def _pad(k, carry):
        rowlist[pl.ds(k * 16, 16)] = jnp.zeros((16,), jnp.int32)
        jlist[pl.ds(k * 16, 16)] = jnp.full((16,), _TW, jnp.int32)
        return carry

    lax.fori_loop(0, _LCAP // 16, _pad, 0)
    pltpu.sync_copy(idx_hbm, idx_v)

    def _bin(k, cur):
        jv = idx_v[pl.ds(k * 16, 16)]
        m = (jv >> 6) == w
        plsc.store_compressed(rowlist.at[pl.ds(cur, 16)], k * 16 + lanes,
                              mask=m)
        plsc.store_compressed(jlist.at[pl.ds(cur, 16)], jv & (_TW - 1),
                              mask=m)
        cnt = plsc.all_reduce_population_count(m)
        return cur + cnt[0]

    cur = lax.fori_loop(0, _TK // 16, _bin, jnp.int32(0))
    nch2 = (cur + 2 * _CHK - 1) // (2 * _CHK)

    def _issue(c, buf):
        return pltpu.async_copy(
            y_hbm.at[rowlist.at[pl.ds(c * _CHK, _CHK)]], yb.at[buf],
            sems[buf])

    def _wait(buf):
        pltpu.make_async_copy(
            y_hbm.at[rowlist.at[pl.ds(0, _CHK)]], yb.at[buf],
            sems[buf]).wait()

    def _process(c, buf):
        jv = jlist[pl.ds(c * _CHK, 16)]
        for i in range(16):
            j = jv[i]
            for u in range(_D // 16):
                sl = pl.ds(u * 16, 16)
                acc[j, sl] = acc[j, sl] + yb[buf, i, sl]

    _issue(0, 0)

    def _pair(k2, carry):
        c = k2 * 2
        _wait(0)
        _issue(c + 1, 1)
        _process(c, 0)
        _wait(1)
        _issue(c + 2, 0)
        _process(c + 1, 1)
        return carry

    lax.fori_loop(0, nch2, _pair, 0)
    _wait(0)
    pltpu.sync_copy(acc.at[pl.ds(0, _TW)], out_hbm.at[pl.ds(w * _TW, _TW)])


def kernel(offsets, jagged, weight, bias, index, weight_p, weight_out,
           reverse_index, gates, gates_index, bias_p, bias_out):
    g = _gather_gates(gates.reshape(-1), gates_index)
    y = _mlp(offsets, jagged, weight, bias, weight_p, bias_p, weight_out,
             bias_out, g.reshape(_TK, 1))
    return _scatter_combine(y, index)


# speedup vs baseline: 1.0304x; 1.0304x over previous
import functools

import jax
import jax.numpy as jnp
from jax import lax
from jax.experimental import pallas as pl
from jax.experimental.pallas import tpu as pltpu
from jax.experimental.pallas import tpu_sc as plsc

_E = 8
_T = 2048
_K = 2
_TK = _T * _K
_D = 1024
_F = 2048
_SEG = _TK // _E
_BF = 2048
_NF = _F // _BF

_NC = 2
_NS = 16
_DH = _D // _NC
_RPT = _TK // _NS
_RB = 64
_NB = _RPT // _RB
_WPT = _T // _NS
_GPW = _TK // (_NC * _NS)

_MESH = plsc.VectorSubcoreMesh(core_axis_name="c", subcore_axis_name="s")


@functools.partial(
    pl.kernel,
    out_type=jax.ShapeDtypeStruct((_TK,), jnp.float32),
    mesh=_MESH,
    scratch_types=[
        pltpu.VMEM((_TK,), jnp.float32),
        pltpu.VMEM((_GPW,), jnp.int32),
        pltpu.VMEM((_GPW,), jnp.float32),
    ],
    compiler_params=pltpu.CompilerParams(needs_layout_passes=False),
)
def _gather_gates(gates_hbm, gidx_hbm, g_hbm, gates_v, gidx_v, gout_v):
    wid = lax.axis_index("s") * _NC + lax.axis_index("c")
    base = wid * _GPW
    pltpu.sync_copy(gates_hbm, gates_v)
    pltpu.sync_copy(gidx_hbm.at[pl.ds(base, _GPW)], gidx_v)
    for i in range(_GPW // 16):
        idx16 = gidx_v[pl.ds(i * 16, 16)]
        gout_v[pl.ds(i * 16, 16)] = plsc.load_gather(gates_v, [idx16])
    pltpu.sync_copy(gout_v, g_hbm.at[pl.ds(base, _GPW)])


def _mlp_body(offs_ref, x_ref, w_ref, b_ref, wp_ref, bp_ref, wo_ref, bo_ref,
              g_ref, y_ref):
    f = pl.program_id(1)
    x = x_ref[...].astype(jnp.bfloat16)
    xw = jnp.dot(x, w_ref[0].astype(jnp.bfloat16),
                 preferred_element_type=jnp.float32) + b_ref[0, 0]
    xwp = jnp.dot(x, wp_ref[0].astype(jnp.bfloat16),
                  preferred_element_type=jnp.float32) + bp_ref[0, 0]
    h = ((xw * jax.nn.sigmoid(xw)) * xwp).astype(jnp.bfloat16)
    part = jnp.dot(h, wo_ref[0].astype(jnp.bfloat16),
                   preferred_element_type=jnp.float32)

    @pl.when(f == 0)
    def _():
        y_ref[...] = part

    @pl.when(f > 0)
    def _():
        y_ref[...] = y_ref[...] + part

    @pl.when(f == _NF - 1)
    def _():
        y_ref[...] = (y_ref[...] + bo_ref[0, 0]) * g_ref[...]


def _mlp(offsets, jagged, weight, bias, weight_p, bias_p, weight_out,
         bias_out, g2d):
    grid_spec = pltpu.PrefetchScalarGridSpec(
        num_scalar_prefetch=1,
        grid=(_E, _NF),
        in_specs=[
            pl.BlockSpec((_SEG, _D), lambda e, f, offs: (offs[e] // _SEG, 0)),
            pl.BlockSpec((1, _D, _BF), lambda e, f, offs: (e, 0, f)),
            pl.BlockSpec((1, 1, _BF), lambda e, f, offs: (e, 0, f)),
            pl.BlockSpec((1, _D, _BF), lambda e, f, offs: (e, 0, f)),
            pl.BlockSpec((1, 1, _BF), lambda e, f, offs: (e, 0, f)),
            pl.BlockSpec((1, _BF, _D), lambda e, f, offs: (e, f, 0)),
            pl.BlockSpec((1, 1, _D), lambda e, f, offs: (e, 0, 0)),
            pl.BlockSpec((_SEG, 1), lambda e, f, offs: (e, 0)),
        ],
        out_specs=pl.BlockSpec((_SEG, _D), lambda e, f, offs: (e, 0)),
    )
    return pl.pallas_call(
        _mlp_body,
        grid_spec=grid_spec,
        out_shape=jax.ShapeDtypeStruct((_TK, _D), jnp.float32),
        compiler_params=pltpu.CompilerParams(
            dimension_semantics=("arbitrary", "arbitrary"),
            vmem_limit_bytes=100 * 1024 * 1024),
    )(offsets, jagged, weight, bias.reshape(_E, 1, _F), weight_p,
      bias_p.reshape(_E, 1, _F), weight_out, bias_out.reshape(_E, 1, _D), g2d)


_NW = _NC * _NS
_TW = _T // _NW
_CHK = 16
_LCAP = _TK + 4 * _CHK


@functools.partial(
    pl.kernel,
    out_type=jax.ShapeDtypeStruct((_T, _D), jnp.float32),
    mesh=_MESH,
    scratch_types=[
        pltpu.VMEM((_TW + 1, _D), jnp.float32),
        pltpu.VMEM((_TK,), jnp.int32),
        pltpu.VMEM((_LCAP,), jnp.int32),
        pltpu.VMEM((_LCAP,), jnp.int32),
        pltpu.VMEM((2, _CHK, _D), jnp.float32),
        pltpu.SemaphoreType.DMA,
        pltpu.SemaphoreType.DMA,
    ],
    compiler_params=pltpu.CompilerParams(
        needs_layout_passes=False, use_tc_tiling_on_sc=True),
)
def _scatter_combine(y_hbm, idx_hbm, out_hbm, acc, idx_v, rowlist, jlist, yb,
                     semA, semB):
    w = lax.axis_index("s") * _NC + lax.axis_index("c")
    lanes = jnp.arange(16, dtype=jnp.int32)
    sems = (semA, semB)

    def _zrow(r, carry):
        for u in range(_D // 16):
            acc[r, pl.ds(u * 16, 16)] = jnp.zeros((16,), jnp.float32)
        return carry

    lax.fori_loop(0, _TW + 1, _zrow, 0)

    def _pad(k, carry):
        rowlist[pl.ds(k * 16, 16)] = jnp.zeros((16,), jnp.int32)
        jlist[pl.ds(k * 16, 16)] = jnp.full((16,), _TW, jnp.int32)
        return carry

    lax.fori_loop(0, _LCAP // 16, _pad, 0)
    pltpu.sync_copy(idx_hbm, idx_v)

    def _bin(k, cur):
        jv = idx_v[pl.ds(k * 16, 16)]
        m = (jv >> 6) == w
        plsc.store_compressed(rowlist.at[pl.ds(cur, 16)], k * 16 + lanes,
                              mask=m)
        plsc.store_compressed(jlist.at[pl.ds(cur, 16)], jv & (_TW - 1),
                              mask=m)
        cnt = plsc.all_reduce_population_count(m)
        return cur + cnt[0]

    cur = lax.fori_loop(0, _TK // 16, _bin, jnp.int32(0))
    nch2 = (cur + 2 * _CHK - 1) // (2 * _CHK)

    def _issue(c, buf):
        return pltpu.async_copy(
            y_hbm.at[rowlist.at[pl.ds(c * _CHK, _CHK)]], yb.at[buf],
            sems[buf])

    def _wait(buf):
        pltpu.make_async_copy(
            y_hbm.at[rowlist.at[pl.ds(0, _CHK)]], yb.at[buf],
            sems[buf]).wait()

    def _process(c, buf):
        jv = jlist[pl.ds(c * _CHK, 16)]
        for i in range(16):
            j = jv[i]
            for u in range(_D // 16):
                sl = pl.ds(u * 16, 16)
                plsc.addupdate(acc.at[j, sl], yb[buf, i, sl])

    _issue(0, 0)

    def _pair(k2, carry):
        c = k2 * 2
        _wait(0)
        _issue(c + 1, 1)
        _process(c, 0)
        _wait(1)
        _issue(c + 2, 0)
        _process(c + 1, 1)
        return carry

    lax.fori_loop(0, nch2, _pair, 0)
    _wait(0)
    pltpu.sync_copy(acc.at[pl.ds(0, _TW)], out_hbm.at[pl.ds(w * _TW, _TW)])


def kernel(offsets, jagged, weight, bias, index, weight_p, weight_out,
           reverse_index, gates, gates_index, bias_p, bias_out):
    g = _gather_gates(gates.reshape(-1), gates_index)
    y = _mlp(offsets, jagged, weight, bias, weight_p, bias_p, weight_out,
             bias_out, g.reshape(_TK, 1))
    return _scatter_combine(y, index)
